# Initial kernel scaffold; baseline (speedup 1.0000x reference)
import jax
import jax.numpy as jnp
from jax.experimental import pallas as pl


def kernel(conv1_w, conv1_b, conv2_w, conv2_b, fc1_w, fc1_b, fc2_w, fc2_b, x):
    raise NotImplementedError("write your pallas kernel here")


# speedup vs baseline: 48.2087x; 48.2087x over previous
import jax
import jax.numpy as jnp
from jax.experimental import pallas as pl
from jax.experimental.pallas import tpu as pltpu


_GROUP = 1024


def _conv_stack_kernel(x_ref, c1w_ref, c1b_ref, c2w_ref, c2b_ref, o_ref,
                       xp_ref, p1_ref):
    xp_ref[...] = jnp.zeros((30, 30, 8, 128), jnp.float32)
    xp_ref[1:29, 1:29] = x_ref[...]

    def c1_body(c, carry):
        acc = c1w_ref[c, 0] * xp_ref[0:28, 0:28]
        for t in range(1, 9):
            ky, kx = t // 3, t % 3
            acc = acc + c1w_ref[c, t] * xp_ref[ky:ky + 28, kx:kx + 28]
        y = jnp.maximum(acc + c1b_ref[c], 0.0)
        y = y.reshape(14, 2, 14, 2, 8, 128)
        p = jnp.max(jnp.max(y, axis=3), axis=1)
        p1_ref[c] = jnp.pad(p, ((1, 1), (1, 1), (0, 0), (0, 0)))
        return carry

    jax.lax.fori_loop(0, 4, c1_body, 0)

    def c2_body(co, carry):
        acc = c2w_ref[co, 0] * p1_ref[0, 0:14, 0:14]
        for k in range(1, 36):
            ci, t = k // 9, k % 9
            ky, kx = t // 3, t % 3
            acc = acc + c2w_ref[co, k] * p1_ref[ci, ky:ky + 14, kx:kx + 14]
        y = jnp.maximum(acc + c2b_ref[co], 0.0)
        y = y.reshape(7, 2, 7, 2, 8, 128)
        p = jnp.max(jnp.max(y, axis=3), axis=1)
        o_ref[pl.ds(co * 49, 49)] = p.reshape(49, 8, 128)
        return carry

    jax.lax.fori_loop(0, 16, c2_body, 0)


def _mlp_kernel(a_ref, w1_ref, b1_ref, w2_ref, b2_ref, o_ref):
    a = a_ref[...].astype(jnp.bfloat16)
    h = jnp.dot(w1_ref[...], a, preferred_element_type=jnp.float32)
    h = jnp.maximum(h + b1_ref[...], 0.0).astype(jnp.bfloat16)
    o = jnp.dot(w2_ref[...], h, preferred_element_type=jnp.float32)
    o_ref[...] = o + b2_ref[...]


def kernel(conv1_w, conv1_b, conv2_w, conv2_b, fc1_w, fc1_b, fc2_w, fc2_b, x):
    B = x.shape[0]
    Bp = ((B + _GROUP - 1) // _GROUP) * _GROUP
    xf = x.reshape(B, 784)
    if Bp != B:
        xf = jnp.pad(xf, ((0, Bp - B), (0, 0)))
    xg = xf.T.reshape(28, 28, Bp // 128, 128)

    c1w = conv1_w.reshape(4, 9)
    c2w = conv2_w.reshape(16, 36)

    act = pl.pallas_call(
        _conv_stack_kernel,
        out_shape=jax.ShapeDtypeStruct((784, Bp // 128, 128), jnp.float32),
        grid=(Bp // _GROUP,),
        in_specs=[
            pl.BlockSpec((28, 28, 8, 128), lambda i: (0, 0, i, 0)),
            pl.BlockSpec(memory_space=pltpu.SMEM),
            pl.BlockSpec(memory_space=pltpu.SMEM),
            pl.BlockSpec(memory_space=pltpu.SMEM),
            pl.BlockSpec(memory_space=pltpu.SMEM),
        ],
        out_specs=pl.BlockSpec((784, 8, 128), lambda i: (0, i, 0)),
        scratch_shapes=[
            pltpu.VMEM((30, 30, 8, 128), jnp.float32),
            pltpu.VMEM((4, 16, 16, 8, 128), jnp.float32),
        ],
        compiler_params=pltpu.CompilerParams(
            dimension_semantics=("parallel",),
            vmem_limit_bytes=48 * 1024 * 1024,
        ),
    )(xg, c1w, conv1_b, c2w, conv2_b)

    act2 = act.reshape(784, Bp)

    TB = 1024
    w1t = fc1_w.T.astype(jnp.bfloat16)
    w2t = fc2_w.T.astype(jnp.bfloat16)
    b1c = fc1_b.reshape(400, 1)
    b2c = fc2_b.reshape(10, 1)

    out = pl.pallas_call(
        _mlp_kernel,
        out_shape=jax.ShapeDtypeStruct((10, Bp), jnp.float32),
        grid=(Bp // TB,),
        in_specs=[
            pl.BlockSpec((784, TB), lambda i: (0, i)),
            pl.BlockSpec((400, 784), lambda i: (0, 0)),
            pl.BlockSpec((400, 1), lambda i: (0, 0)),
            pl.BlockSpec((10, 400), lambda i: (0, 0)),
            pl.BlockSpec((10, 1), lambda i: (0, 0)),
        ],
        out_specs=pl.BlockSpec((10, TB), lambda i: (0, i)),
        compiler_params=pltpu.CompilerParams(
            dimension_semantics=("parallel",),
            vmem_limit_bytes=32 * 1024 * 1024,
        ),
    )(act2, w1t, b1c, w2t, b2c)

    return out[:, :B].T


# speedup vs baseline: 52.2609x; 1.0841x over previous
import jax
import jax.numpy as jnp
from jax.experimental import pallas as pl
from jax.experimental.pallas import tpu as pltpu


_GROUP = 1024


def _fused_kernel(x_ref, c1w_ref, c1b_ref, c2w_ref, c2b_ref,
                  w1_ref, b1_ref, w2_ref, b2_ref, o_ref,
                  xp_ref, p1_ref, act_ref):
    xp_ref[...] = jnp.zeros((30, 30, 8, 128), jnp.float32)
    xp_ref[1:29, 1:29] = x_ref[...]

    def c1_body(c, carry):
        acc = c1w_ref[c, 0] * xp_ref[0:28, 0:28]
        for t in range(1, 9):
            ky, kx = t // 3, t % 3
            acc = acc + c1w_ref[c, t] * xp_ref[ky:ky + 28, kx:kx + 28]
        y = jnp.maximum(acc + c1b_ref[c], 0.0)
        y = y.reshape(14, 2, 14, 2, 8, 128)
        p = jnp.max(jnp.max(y, axis=3), axis=1)
        p1_ref[c] = jnp.pad(p, ((1, 1), (1, 1), (0, 0), (0, 0)))
        return carry

    jax.lax.fori_loop(0, 4, c1_body, 0)

    def c2_body(co, carry):
        acc = c2w_ref[co, 0] * p1_ref[0, 0:14, 0:14]
        for k in range(1, 36):
            ci, t = k // 9, k % 9
            ky, kx = t // 3, t % 3
            acc = acc + c2w_ref[co, k] * p1_ref[ci, ky:ky + 14, kx:kx + 14]
        y = jnp.maximum(acc + c2b_ref[co], 0.0)
        y = y.reshape(7, 2, 7, 2, 8, 128)
        p = jnp.max(jnp.max(y, axis=3), axis=1)
        act_ref[pl.ds(co * 49, 49)] = p.reshape(49, 8, 128)
        return carry

    jax.lax.fori_loop(0, 16, c2_body, 0)

    a = act_ref[...].reshape(784, 1024).astype(jnp.bfloat16)
    h = jnp.dot(w1_ref[...], a, preferred_element_type=jnp.float32)
    h = jnp.maximum(h + b1_ref[...], 0.0).astype(jnp.bfloat16)
    o = jnp.dot(w2_ref[...], h, preferred_element_type=jnp.float32)
    o_ref[...] = o + b2_ref[...]


def kernel(conv1_w, conv1_b, conv2_w, conv2_b, fc1_w, fc1_b, fc2_w, fc2_b, x):
    B = x.shape[0]
    Bp = ((B + _GROUP - 1) // _GROUP) * _GROUP
    xf = x.reshape(B, 784)
    if Bp != B:
        xf = jnp.pad(xf, ((0, Bp - B), (0, 0)))
    xg = xf.T.reshape(28, 28, Bp // 128, 128)

    c1w = conv1_w.reshape(4, 9)
    c2w = conv2_w.reshape(16, 36)
    w1t = fc1_w.T.astype(jnp.bfloat16)
    w2t = fc2_w.T.astype(jnp.bfloat16)
    b1c = fc1_b.reshape(400, 1)
    b2c = fc2_b.reshape(10, 1)

    out = pl.pallas_call(
        _fused_kernel,
        out_shape=jax.ShapeDtypeStruct((10, Bp), jnp.float32),
        grid=(Bp // _GROUP,),
        in_specs=[
            pl.BlockSpec((28, 28, 8, 128), lambda i: (0, 0, i, 0)),
            pl.BlockSpec(memory_space=pltpu.SMEM),
            pl.BlockSpec(memory_space=pltpu.SMEM),
            pl.BlockSpec(memory_space=pltpu.SMEM),
            pl.BlockSpec(memory_space=pltpu.SMEM),
            pl.BlockSpec((400, 784), lambda i: (0, 0)),
            pl.BlockSpec((400, 1), lambda i: (0, 0)),
            pl.BlockSpec((10, 400), lambda i: (0, 0)),
            pl.BlockSpec((10, 1), lambda i: (0, 0)),
        ],
        out_specs=pl.BlockSpec((10, _GROUP), lambda i: (0, i)),
        scratch_shapes=[
            pltpu.VMEM((30, 30, 8, 128), jnp.float32),
            pltpu.VMEM((4, 16, 16, 8, 128), jnp.float32),
            pltpu.VMEM((784, 8, 128), jnp.float32),
        ],
        compiler_params=pltpu.CompilerParams(
            dimension_semantics=("parallel",),
            vmem_limit_bytes=48 * 1024 * 1024,
        ),
    )(xg, c1w, conv1_b, c2w, conv2_b, w1t, b1c, w2t, b2c)

    return out[:, :B].T
